# baseline (speedup 1.0000x reference)
import functools

import jax
import jax.numpy as jnp
from jax import lax
from jax.experimental import pallas as pl
from jax.experimental.pallas import tpu as pltpu
from jax.experimental.pallas import tpu_sc as plsc

_NUM_CORES = 2
_NUM_SUBCORES = 16
_NUM_WORKERS = _NUM_CORES * _NUM_SUBCORES
_CHUNK = 128


@functools.partial(jax.jit, static_argnums=(2, 3))
def _gather_rows(idx2d, table, num_idx, dim):
    chunks_per_w = num_idx // (_NUM_WORKERS * _CHUNK)
    mesh = plsc.VectorSubcoreMesh(core_axis_name="c", subcore_axis_name="s")

    @functools.partial(
        pl.kernel,
        mesh=mesh,
        out_type=jax.ShapeDtypeStruct((num_idx, dim), jnp.float32),
        scratch_types=[
            pltpu.VMEM((chunks_per_w, _CHUNK), jnp.int32),
            pltpu.VMEM((_CHUNK, dim), jnp.float32),
            pltpu.SemaphoreType.DMA,
        ],
        compiler_params=pltpu.CompilerParams(use_tc_tiling_on_sc=False),
    )
    def gather_kernel(idx_hbm, table_hbm, out_hbm, idx_v, rows_v, sem):
        wid = lax.axis_index("s") * _NUM_CORES + lax.axis_index("c")
        pltpu.sync_copy(idx_hbm.at[pl.ds(wid * chunks_per_w, chunks_per_w)], idx_v)
        base = wid * chunks_per_w * _CHUNK

        def body(j, carry):
            pltpu.async_copy(table_hbm.at[idx_v.at[j]], rows_v, sem).wait()
            pltpu.sync_copy(rows_v, out_hbm.at[pl.ds(base + j * _CHUNK, _CHUNK)])
            return carry

        lax.fori_loop(0, chunks_per_w, body, 0)

    return gather_kernel(idx2d, table)


def kernel(token_ids, embedding_table):
    b, s = token_ids.shape
    num_idx = b * s
    dim = embedding_table.shape[1]
    idx2d = token_ids.reshape(num_idx // _CHUNK, _CHUNK).astype(jnp.int32)
    out = _gather_rows(idx2d, embedding_table, num_idx, dim)
    return out.reshape(b, s, dim)


# speedup vs baseline: 1.0734x; 1.0734x over previous
import functools

import jax
import jax.numpy as jnp
from jax import lax
from jax.experimental import pallas as pl
from jax.experimental.pallas import tpu as pltpu
from jax.experimental.pallas import tpu_sc as plsc

_NUM_CORES = 2
_NUM_SUBCORES = 16
_NUM_WORKERS = _NUM_CORES * _NUM_SUBCORES
_CHUNK = 128
_NBUF = 8


@functools.partial(jax.jit, static_argnums=(2, 3))
def _gather_rows(idx2d, table, num_idx, dim):
    chunks_per_w = num_idx // (_NUM_WORKERS * _CHUNK)
    n_outer = chunks_per_w // _NBUF
    mesh = plsc.VectorSubcoreMesh(core_axis_name="c", subcore_axis_name="s")

    @functools.partial(
        pl.kernel,
        mesh=mesh,
        out_type=jax.ShapeDtypeStruct((num_idx, dim), jnp.float32),
        scratch_types=[
            pltpu.VMEM((chunks_per_w, _CHUNK), jnp.int32),
            [pltpu.VMEM((_CHUNK, dim), jnp.float32) for _ in range(_NBUF)],
            [pltpu.SemaphoreType.DMA for _ in range(_NBUF)],
            [pltpu.SemaphoreType.DMA for _ in range(_NBUF)],
        ],
        compiler_params=pltpu.CompilerParams(use_tc_tiling_on_sc=False),
    )
    def gather_kernel(idx_hbm, table_hbm, out_hbm, idx_v, rows, gsem, wsem):
        wid = lax.axis_index("s") * _NUM_CORES + lax.axis_index("c")
        pltpu.sync_copy(idx_hbm.at[pl.ds(wid * chunks_per_w, chunks_per_w)], idx_v)
        base = wid * chunks_per_w * _CHUNK

        for b in range(_NBUF):
            pltpu.async_copy(table_hbm.at[idx_v.at[b]], rows[b], gsem[b])

        def outer(g, carry):
            for b in range(_NBUF):
                j = g * _NBUF + b
                pltpu.make_async_copy(table_hbm.at[idx_v.at[j]], rows[b], gsem[b]).wait()
                pltpu.async_copy(
                    rows[b], out_hbm.at[pl.ds(base + j * _CHUNK, _CHUNK)], wsem[b]
                )

            @pl.when(g + 1 < n_outer)
            def _():
                for b in range(_NBUF):
                    jn = (g + 1) * _NBUF + b
                    pltpu.make_async_copy(
                        rows[b], out_hbm.at[pl.ds(base, _CHUNK)], wsem[b]
                    ).wait()
                    pltpu.async_copy(table_hbm.at[idx_v.at[jn]], rows[b], gsem[b])

            return carry

        lax.fori_loop(0, n_outer, outer, 0)

        for b in range(_NBUF):
            pltpu.make_async_copy(rows[b], out_hbm.at[pl.ds(base, _CHUNK)], wsem[b]).wait()

    return gather_kernel(idx2d, table)


def kernel(token_ids, embedding_table):
    b, s = token_ids.shape
    num_idx = b * s
    dim = embedding_table.shape[1]
    idx2d = token_ids.reshape(num_idx // _CHUNK, _CHUNK).astype(jnp.int32)
    out = _gather_rows(idx2d, embedding_table, num_idx, dim)
    return out.reshape(b, s, dim)
